# baseline (speedup 1.0000x reference)
import functools

import jax
import jax.numpy as jnp
from jax import lax
from jax.experimental import pallas as pl
from jax.experimental.pallas import tpu as pltpu
from jax.experimental.pallas import tpu_sc as plsc

POS_OFFSET = 2
NC = 2
NS = 16
NW = NC * NS
LANES = 16
CH = 8


@functools.partial(jax.jit, static_argnums=(4, 5))
def _embed_lookup(ids2, pos2, embed_tokens, embed_positions, n_tokens, d):
    rpw = n_tokens // NW
    nch = rpw // CH
    wps = ids2.shape[1] // rpw
    mesh = plsc.VectorSubcoreMesh(
        core_axis_name="c", subcore_axis_name="s",
        num_cores=NC, num_subcores=NS)

    @functools.partial(
        pl.kernel,
        out_type=jax.ShapeDtypeStruct((n_tokens, d), jnp.float32),
        mesh=mesh,
        scratch_types=[
            pltpu.VMEM((rpw,), jnp.int32),
            pltpu.VMEM((rpw,), jnp.int32),
            pltpu.VMEM((2, CH, d), jnp.float32),
            pltpu.VMEM((2, CH, d), jnp.float32),
            pltpu.VMEM((2, CH, d), jnp.float32),
            [pltpu.SemaphoreType.DMA] * 2,
            [pltpu.SemaphoreType.DMA] * 2,
            [pltpu.SemaphoreType.DMA] * 2,
        ],
    )
    def body(ids_hbm, pos_hbm, tok_tab, pos_tab, out_hbm,
             idx_t, idx_p, buf_t, buf_p, buf_o, sems_t, sems_p, sems_o):
        wid = lax.axis_index("s") * NC + lax.axis_index("c")
        brow = wid // wps
        bcol = (wid % wps) * rpw
        pltpu.sync_copy(ids_hbm.at[brow, pl.ds(bcol, rpw)], idx_t)
        pltpu.sync_copy(pos_hbm.at[brow, pl.ds(bcol, rpw)], idx_p)

        @pl.loop(0, rpw // LANES)
        def _(j):
            sl = pl.ds(pl.multiple_of(j * LANES, LANES), LANES)
            idx_p[sl] = idx_p[sl] + POS_OFFSET

        base = wid * rpw

        def g_desc(c, slot):
            off = pl.multiple_of(c * CH, CH)
            ct = pltpu.make_async_copy(
                tok_tab.at[idx_t.at[pl.ds(off, CH)]],
                buf_t.at[slot], sems_t[slot])
            cp = pltpu.make_async_copy(
                pos_tab.at[idx_p.at[pl.ds(off, CH)]],
                buf_p.at[slot], sems_p[slot])
            return ct, cp

        def o_desc(c, slot):
            off = pl.multiple_of(c * CH, CH)
            return pltpu.make_async_copy(
                buf_o.at[slot], out_hbm.at[pl.ds(base + off, CH)],
                sems_o[slot])

        def issue_g(c, slot):
            ct, cp = g_desc(c, slot)
            ct.start()
            cp.start()

        def add_chunk(slot):
            @pl.loop(0, d // (2 * LANES))
            def _(i):
                o1 = pl.ds(pl.multiple_of(i * 2 * LANES, LANES), LANES)
                o2 = pl.ds(pl.multiple_of(i * 2 * LANES + LANES, LANES),
                           LANES)
                for r in range(CH):
                    buf_o[slot, r, o1] = (
                        buf_t[slot, r, o1] + buf_p[slot, r, o1])
                    buf_o[slot, r, o2] = (
                        buf_t[slot, r, o2] + buf_p[slot, r, o2])

        issue_g(0, 0)
        issue_g(1, 1)

        def step(c, slot, first, last):
            ct, cp = g_desc(c, slot)
            ct.wait()
            cp.wait()

            @pl.when(jnp.logical_not(first))
            def _():
                o_desc(c - 2, slot).wait()

            add_chunk(slot)
            o_desc(c, slot).start()

            @pl.when(jnp.logical_not(last))
            def _():
                issue_g(c + 2, slot)

        @pl.loop(0, nch // 2)
        def _(i):
            a = pl.multiple_of(i * 2, 2)
            step(a, 0, i == 0, i == nch // 2 - 1)
            step(a + 1, 1, i == 0, i == nch // 2 - 1)

        o_desc(nch - 2, 0).wait()
        o_desc(nch - 1, 1).wait()

    return body(ids2, pos2, embed_tokens, embed_positions)


def kernel(input_ids, positions, embed_tokens, embed_positions):
    b, s = input_ids.shape
    d = embed_tokens.shape[1]
    n = b * s
    out = _embed_lookup(input_ids, positions, embed_tokens, embed_positions,
                        n, d)
    return out.reshape(b, s, d)


# speedup vs baseline: 1.1266x; 1.1266x over previous
import functools

import jax
import jax.numpy as jnp
from jax import lax
from jax.experimental import pallas as pl
from jax.experimental.pallas import tpu as pltpu
from jax.experimental.pallas import tpu_sc as plsc

POS_OFFSET = 2
NC = 2
NS = 16
NW = NC * NS
LANES = 16
CH = 8


@functools.partial(jax.jit, static_argnums=(4, 5))
def _embed_lookup(ids2, pos2, embed_tokens, embed_positions, n_tokens, d):
    rpw = n_tokens // NW
    nch = rpw // CH
    wps = ids2.shape[1] // rpw
    mesh = plsc.VectorSubcoreMesh(
        core_axis_name="c", subcore_axis_name="s",
        num_cores=NC, num_subcores=NS)

    @functools.partial(
        pl.kernel,
        out_type=jax.ShapeDtypeStruct((n_tokens, d), jnp.float32),
        mesh=mesh,
        scratch_types=[
            pltpu.VMEM((rpw,), jnp.int32),
            pltpu.VMEM((rpw,), jnp.int32),
            pltpu.VMEM((2, CH, d), jnp.float32),
            pltpu.VMEM((2, CH, d), jnp.float32),
            pltpu.VMEM((2, CH, d), jnp.float32),
            [pltpu.SemaphoreType.DMA] * 2,
            [pltpu.SemaphoreType.DMA] * 2,
            [pltpu.SemaphoreType.DMA] * 2,
        ],
    )
    def body(ids_hbm, pos_hbm, tok_tab, pos_tab, out_hbm,
             idx_t, idx_p, buf_t, buf_p, buf_o, sems_t, sems_p, sems_o):
        wid = lax.axis_index("s") * NC + lax.axis_index("c")
        brow = wid // wps
        bcol = (wid % wps) * rpw
        pltpu.sync_copy(ids_hbm.at[brow, pl.ds(bcol, rpw)], idx_t)
        pltpu.sync_copy(pos_hbm.at[brow, pl.ds(bcol, rpw)], idx_p)

        @pl.loop(0, rpw // LANES)
        def _(j):
            sl = pl.ds(pl.multiple_of(j * LANES, LANES), LANES)
            idx_p[sl] = idx_p[sl] + POS_OFFSET

        base = wid * rpw

        def g_desc(c, slot):
            off = pl.multiple_of(c * CH, CH)
            ct = pltpu.make_async_copy(
                tok_tab.at[idx_t.at[pl.ds(off, CH)]],
                buf_t.at[slot], sems_t[slot])
            cp = pltpu.make_async_copy(
                pos_tab.at[idx_p.at[pl.ds(off, CH)]],
                buf_p.at[slot], sems_p[slot])
            return ct, cp

        def o_desc(c, slot):
            off = pl.multiple_of(c * CH, CH)
            return pltpu.make_async_copy(
                buf_o.at[slot], out_hbm.at[pl.ds(base + off, CH)],
                sems_o[slot])

        def issue_g(c, slot):
            ct, cp = g_desc(c, slot)
            ct.start()
            cp.start()

        def add_chunk(slot):
            @plsc.parallel_loop(0, d // LANES, unroll=2)
            def _(i):
                off = pl.ds(pl.multiple_of(i * LANES, LANES), LANES)
                for r in range(CH):
                    buf_o[slot, r, off] = (
                        buf_t[slot, r, off] + buf_p[slot, r, off])

        issue_g(0, 0)
        issue_g(1, 1)

        def step(c, slot, first, last):
            ct, cp = g_desc(c, slot)
            ct.wait()
            cp.wait()

            @pl.when(jnp.logical_not(first))
            def _():
                o_desc(c - 2, slot).wait()

            add_chunk(slot)
            o_desc(c, slot).start()

            @pl.when(jnp.logical_not(last))
            def _():
                issue_g(c + 2, slot)

        @pl.loop(0, nch // 2)
        def _(i):
            a = pl.multiple_of(i * 2, 2)
            step(a, 0, i == 0, i == nch // 2 - 1)
            step(a + 1, 1, i == 0, i == nch // 2 - 1)

        o_desc(nch - 2, 0).wait()
        o_desc(nch - 1, 1).wait()

    return body(ids2, pos2, embed_tokens, embed_positions)


def kernel(input_ids, positions, embed_tokens, embed_positions):
    b, s = input_ids.shape
    d = embed_tokens.shape[1]
    n = b * s
    out = _embed_lookup(input_ids, positions, embed_tokens, embed_positions,
                        n, d)
    return out.reshape(b, s, d)
